# Initial kernel scaffold; baseline (speedup 1.0000x reference)
import jax
import jax.numpy as jnp
from jax.experimental import pallas as pl


def kernel(gt_points, gt_normals, gt_images, pred_reconst, pred_coord_0, pred_coord_1, pred_coord_2, pred_before_0, pred_before_1, pred_before_2, edges_0, edges_1, edges_2, lap_idx_0, lap_idx_1, lap_idx_2):
    raise NotImplementedError("write your pallas kernel here")


# speedup vs baseline: 1.2137x; 1.2137x over previous
import jax
import jax.numpy as jnp
from jax import lax
from jax.experimental import pallas as pl
from jax.experimental.pallas import tpu as pltpu

_B = 4
_NGT = 2048
_NSL = (156, 618, 2466)
_NEL = (462, 1848, 7392)
_TS = 512
_BIG = 1e9
_EPS = 1e-12
_HI = lax.Precision.HIGHEST

_NORMAL_W = 0.5
_EDGE_W = 0.1
_LAP_W = 0.5
_MOVE_W = 0.1
_CHAMFER_W = (1.0, 1.0, 1.0)
_CHAMFER_OPP_W = 0.55
_RECONST_W = 0.1
_LAP_CONST = (0.2, 1.0, 1.0)


def _ceil_to(n, m):
    return ((n + m - 1) // m) * m


def _pad_last(x, n_pad, val):
    if x.shape[-1] == n_pad:
        return x
    pad = [(0, 0)] * (x.ndim - 1) + [(0, n_pad - x.shape[-1])]
    return jnp.pad(x, pad, constant_values=val)


def _chamfer_body(T, NS, gt_ref, gtn_ref, pred_ref, sd1_ref, sd2_ref,
                  near_ref, dmin_ref):
    b = pl.program_id(0)
    t = pl.program_id(1)
    gt = gt_ref[0]
    pred = pred_ref[0]
    ts = pred.shape[1]
    gn = jnp.sum(gt * gt, axis=0, keepdims=True)
    pn = jnp.sum(pred * pred, axis=0, keepdims=True)
    ones_g = jnp.ones_like(gn)
    ones_p = jnp.ones_like(pn)
    ga = jnp.concatenate([gt, gn, ones_g], axis=0)
    pa = jnp.concatenate([-2.0 * pred, ones_p, pn], axis=0)
    d = lax.dot_general(ga, pa, (((0,), (0,)), ((), ())),
                        precision=_HI, preferred_element_type=jnp.float32)

    dcol = jnp.min(d, axis=1, keepdims=True)

    @pl.when(t == 0)
    def _():
        dmin_ref[...] = dcol

    @pl.when(t != 0)
    def _():
        dmin_ref[...] = jnp.minimum(dmin_ref[...], dcol)

    minv = jnp.min(d, axis=0, keepdims=True)
    iota0 = lax.broadcasted_iota(jnp.int32, d.shape, 0)
    idxm = jnp.min(jnp.where(d == minv, iota0, _NGT), axis=0, keepdims=True)
    onehot = (iota0 == idxm).astype(jnp.float32)
    near = lax.dot_general(gtn_ref[0], onehot, (((1,), (0,)), ((), ())),
                           precision=_HI, preferred_element_type=jnp.float32)
    near_ref[0] = near

    lane = lax.broadcasted_iota(jnp.int32, (1, ts), 1) + t * ts
    s2 = jnp.sum(jnp.where(lane < NS, minv, 0.0))

    first = jnp.logical_and(b == 0, t == 0)

    @pl.when(first)
    def _():
        sd2_ref[0, 0] = s2

    @pl.when(jnp.logical_not(first))
    def _():
        sd2_ref[0, 0] = sd2_ref[0, 0] + s2

    @pl.when(jnp.logical_and(t == T - 1, b == 0))
    def _():
        sd1_ref[0, 0] = jnp.sum(dmin_ref[...])

    @pl.when(jnp.logical_and(t == T - 1, b != 0))
    def _():
        sd1_ref[0, 0] = sd1_ref[0, 0] + jnp.sum(dmin_ref[...])


def _chamfer(gt_t, gtn_t, pred_t, NS):
    ns_pad = pred_t.shape[-1]
    T = ns_pad // _TS
    import functools
    body = functools.partial(_chamfer_body, T, NS)
    return pl.pallas_call(
        body,
        grid=(_B, T),
        in_specs=[
            pl.BlockSpec((1, 3, _NGT), lambda b, t: (b, 0, 0)),
            pl.BlockSpec((1, 3, _NGT), lambda b, t: (b, 0, 0)),
            pl.BlockSpec((1, 3, _TS), lambda b, t: (b, 0, t)),
        ],
        out_specs=[
            pl.BlockSpec(memory_space=pltpu.SMEM),
            pl.BlockSpec(memory_space=pltpu.SMEM),
            pl.BlockSpec((1, 3, _TS), lambda b, t: (b, 0, t)),
        ],
        out_shape=[
            jax.ShapeDtypeStruct((1, 1), jnp.float32),
            jax.ShapeDtypeStruct((1, 1), jnp.float32),
            jax.ShapeDtypeStruct((_B, 3, ns_pad), jnp.float32),
        ],
        scratch_shapes=[pltpu.VMEM((_NGT, 1), jnp.float32)],
    )(gt_t, gtn_t, pred_t)


def _edge_body(NS_pad, pred_ref, near_ref, e0_ref, e1_ref, es_ref, cs_ref):
    t = pl.program_id(0)
    e0 = e0_ref[...]
    e1 = e1_ref[...]
    ts = e0.shape[1]
    iota_u = lax.broadcasted_iota(jnp.int32, (NS_pad, ts), 0)
    g0 = (iota_u == e0).astype(jnp.float32)
    g1 = (iota_u == e1).astype(jnp.float32)
    gd = g0 - g1
    acc_e = jnp.float32(0.0)
    acc_c = jnp.float32(0.0)
    for b in range(_B):
        de = lax.dot_general(pred_ref[b], gd, (((1,), (0,)), ((), ())),
                             precision=_HI, preferred_element_type=jnp.float32)
        nn = lax.dot_general(near_ref[b], g0, (((1,), (0,)), ((), ())),
                             precision=_HI, preferred_element_type=jnp.float32)
        en2 = jnp.sum(de * de, axis=0, keepdims=True)
        nn2 = jnp.sum(nn * nn, axis=0, keepdims=True)
        dotc = jnp.sum(de * nn, axis=0, keepdims=True)
        cos = jnp.abs(dotc) / (jnp.maximum(jnp.sqrt(en2), _EPS)
                               * jnp.maximum(jnp.sqrt(nn2), _EPS))
        acc_e = acc_e + jnp.sum(en2)
        acc_c = acc_c + jnp.sum(cos)

    @pl.when(t == 0)
    def _():
        es_ref[0, 0] = acc_e
        cs_ref[0, 0] = acc_c

    @pl.when(t != 0)
    def _():
        es_ref[0, 0] = es_ref[0, 0] + acc_e
        cs_ref[0, 0] = cs_ref[0, 0] + acc_c


def _edge_normal(pred_t, near, e0, e1):
    ns_pad = pred_t.shape[-1]
    ne_pad = e0.shape[-1]
    T = ne_pad // _TS
    import functools
    body = functools.partial(_edge_body, ns_pad)
    return pl.pallas_call(
        body,
        grid=(T,),
        in_specs=[
            pl.BlockSpec((_B, 3, ns_pad), lambda t: (0, 0, 0)),
            pl.BlockSpec((_B, 3, ns_pad), lambda t: (0, 0, 0)),
            pl.BlockSpec((1, _TS), lambda t: (0, t)),
            pl.BlockSpec((1, _TS), lambda t: (0, t)),
        ],
        out_specs=[
            pl.BlockSpec(memory_space=pltpu.SMEM),
            pl.BlockSpec(memory_space=pltpu.SMEM),
        ],
        out_shape=[
            jax.ShapeDtypeStruct((1, 1), jnp.float32),
            jax.ShapeDtypeStruct((1, 1), jnp.float32),
        ],
    )(pred_t, near, e0, e1)


def _lap_body(NS, NS_pad, pred_ref, bef_ref, lapn_ref, cnt_ref,
              ls_ref, ms_ref):
    t = pl.program_id(0)
    lapn = lapn_ref[...]
    cnt = cnt_ref[...]
    ts = cnt.shape[1]
    iota_u = lax.broadcasted_iota(jnp.int32, (NS_pad, ts), 0)
    a = (iota_u == lapn[0:1, :]).astype(jnp.float32)
    for j in range(1, 8):
        a = a + (iota_u == lapn[j:j + 1, :]).astype(jnp.float32)
    lane = lax.broadcasted_iota(jnp.int32, (1, ts), 1) + t * ts
    mask = lane < NS
    acc_l = jnp.float32(0.0)
    acc_m = jnp.float32(0.0)
    for b in range(_B):
        diffb = bef_ref[b] - pred_ref[b]
        s = lax.dot_general(diffb, a, (((1,), (0,)), ((), ())),
                            precision=_HI, preferred_element_type=jnp.float32)
        difft = (bef_ref[b, :, pl.ds(t * ts, ts)]
                 - pred_ref[b, :, pl.ds(t * ts, ts)])
        lapd = jnp.where(mask, difft - s / cnt, 0.0)
        acc_l = acc_l + jnp.sum(lapd * lapd)
        acc_m = acc_m + jnp.sum(jnp.where(mask, difft, 0.0) ** 2)

    @pl.when(t == 0)
    def _():
        ls_ref[0, 0] = acc_l
        ms_ref[0, 0] = acc_m

    @pl.when(t != 0)
    def _():
        ls_ref[0, 0] = ls_ref[0, 0] + acc_l
        ms_ref[0, 0] = ms_ref[0, 0] + acc_m


def _lap_move(pred_t, bef_t, lapn, cnt, NS):
    ns_pad = pred_t.shape[-1]
    T = ns_pad // _TS
    import functools
    body = functools.partial(_lap_body, NS, ns_pad)
    return pl.pallas_call(
        body,
        grid=(T,),
        in_specs=[
            pl.BlockSpec((_B, 3, ns_pad), lambda t: (0, 0, 0)),
            pl.BlockSpec((_B, 3, ns_pad), lambda t: (0, 0, 0)),
            pl.BlockSpec((8, _TS), lambda t: (0, t)),
            pl.BlockSpec((1, _TS), lambda t: (0, t)),
        ],
        out_specs=[
            pl.BlockSpec(memory_space=pltpu.SMEM),
            pl.BlockSpec(memory_space=pltpu.SMEM),
        ],
        out_shape=[
            jax.ShapeDtypeStruct((1, 1), jnp.float32),
            jax.ShapeDtypeStruct((1, 1), jnp.float32),
        ],
    )(pred_t, bef_t, lapn, cnt)


def _bce_body(gt_ref, p_ref, out_ref):
    p = jnp.clip(p_ref[...], 1e-7, 1.0 - 1e-7)
    gt = gt_ref[...]
    out_ref[0, 0] = jnp.sum(gt * jnp.log(p) + (1.0 - gt) * jnp.log(1.0 - p))


def _bce(gt_img, pred_img):
    return pl.pallas_call(
        _bce_body,
        out_specs=pl.BlockSpec(memory_space=pltpu.SMEM),
        out_shape=jax.ShapeDtypeStruct((1, 1), jnp.float32),
    )(gt_img, pred_img)


def _combine_body(*refs):
    ins = refs[:19]
    outs = refs[19:]
    (sd1_0, sd2_0, es_0, cs_0, ls_0, ms_0,
     sd1_1, sd2_1, es_1, cs_1, ls_1, ms_1,
     sd1_2, sd2_2, es_2, cs_2, ls_2, ms_2, bs) = [r[0, 0] for r in ins]
    sd1 = (sd1_0, sd1_1, sd1_2)
    sd2 = (sd2_0, sd2_1, sd2_2)
    es = (es_0, es_1, es_2)
    cs = (cs_0, cs_1, cs_2)
    ls = (ls_0, ls_1, ls_2)
    ms = (ms_0, ms_1, ms_2)
    chamfer = jnp.float32(0.0)
    edge = jnp.float32(0.0)
    normal = jnp.float32(0.0)
    lap = jnp.float32(0.0)
    move = jnp.float32(0.0)
    for i in range(3):
        ns = _NSL[i]
        ne = _NEL[i]
        chamfer = chamfer + _CHAMFER_W[i] * (
            sd1[i] / (_B * _NGT) + _CHAMFER_OPP_W * sd2[i] / (_B * ns))
        normal = normal + cs[i] / (_B * ne)
        edge = edge + es[i] / (_B * ne)
        lap = lap + _LAP_CONST[i] * ls[i] / (_B * ns)
        if i > 0:
            move = move + _LAP_CONST[i] * ms[i] / (_B * ns)
    image = -bs / (_B * 3 * 224 * 224)
    loss = (chamfer + image * _RECONST_W + _LAP_W * lap + _MOVE_W * move
            + _EDGE_W * edge + _NORMAL_W * normal)
    vals = (loss, image, chamfer, edge, lap, move, normal)
    for r, v in zip(outs, vals):
        r[0, 0] = v


def _combine(scalars):
    return pl.pallas_call(
        _combine_body,
        in_specs=[pl.BlockSpec(memory_space=pltpu.SMEM)] * 19,
        out_specs=[pl.BlockSpec(memory_space=pltpu.SMEM)] * 7,
        out_shape=[jax.ShapeDtypeStruct((1, 1), jnp.float32)] * 7,
    )(*scalars)


def kernel(gt_points, gt_normals, gt_images, pred_reconst,
           pred_coord_0, pred_coord_1, pred_coord_2,
           pred_before_0, pred_before_1, pred_before_2,
           edges_0, edges_1, edges_2,
           lap_idx_0, lap_idx_1, lap_idx_2):
    gt_t = jnp.transpose(gt_points, (0, 2, 1)).astype(jnp.float32)
    gtn_t = jnp.transpose(gt_normals, (0, 2, 1)).astype(jnp.float32)
    preds = (pred_coord_0, pred_coord_1, pred_coord_2)
    befs = (pred_before_0, pred_before_1, pred_before_2)
    edges = (edges_0, edges_1, edges_2)
    laps = (lap_idx_0, lap_idx_1, lap_idx_2)

    scalars = []
    for i in range(3):
        ns = _NSL[i]
        ne = _NEL[i]
        ns_pad = _ceil_to(ns, _TS)
        ne_pad = _ceil_to(ne, _TS)
        pred_t = _pad_last(jnp.transpose(preds[i], (0, 2, 1)), ns_pad, _BIG)
        bef_t = _pad_last(jnp.transpose(befs[i], (0, 2, 1)), ns_pad, _BIG)
        e = edges[i].astype(jnp.int32)
        e0 = _pad_last(e[:, 0][None, :], ne_pad, 0)
        e1 = _pad_last(e[:, 1][None, :], ne_pad, 0)
        li = laps[i].astype(jnp.int32)
        lapn = _pad_last(jnp.transpose(li[:, :8], (1, 0)), ns_pad, -1)
        cnt = _pad_last(li[:, 9].astype(jnp.float32)[None, :], ns_pad, 1.0)

        sd1, sd2, near = _chamfer(gt_t, gtn_t, pred_t, ns)
        es, cs = _edge_normal(pred_t, near, e0, e1)
        ls, ms = _lap_move(pred_t, bef_t, lapn, cnt, ns)
        scalars += [sd1, sd2, es, cs, ls, ms]

    gt_img = jnp.reshape(gt_images, (1176, 512))
    p_img = jnp.reshape(pred_reconst, (1176, 512))
    scalars.append(_bce(gt_img, p_img))

    outs = _combine(scalars)
    return tuple(jnp.reshape(o, ()) for o in outs)


# speedup vs baseline: 3.7910x; 3.1234x over previous
import functools

import jax
import jax.numpy as jnp
from jax import lax
from jax.experimental import pallas as pl
from jax.experimental.pallas import tpu as pltpu
from jax.experimental.pallas import tpu_sc as plsc

_B = 4
_NGT = 2048
_NSL = (156, 618, 2466)
_NEL = (462, 1848, 7392)
_TS = 512
_NSP = (512, 1024, 2560)
_NEP = (512, 2048, 7680)
_NW = 32
_ECH = tuple(n // _NW for n in _NEP)
_VCH = tuple(n // _NW for n in _NSP)
_BIG = 1e9
_EPS = 1e-12

_NORMAL_W = 0.5
_EDGE_W = 0.1
_LAP_W = 0.5
_MOVE_W = 0.1
_CHAMFER_W = (1.0, 1.0, 1.0)
_CHAMFER_OPP_W = 0.55
_RECONST_W = 0.1
_LAP_CONST = (0.2, 1.0, 1.0)


def _pad_last(x, n_pad, val):
    if x.shape[-1] == n_pad:
        return x
    pad = [(0, 0)] * (x.ndim - 1) + [(0, n_pad - x.shape[-1])]
    return jnp.pad(x, pad, constant_values=val)


def _chamfer_body(T, NS, gt_ref, pred_ref, sd1_ref, sd2_ref, idx_ref,
                  dmin_ref):
    b = pl.program_id(0)
    t = pl.program_id(1)
    gx = gt_ref[0, :, 0:1]
    gy = gt_ref[0, :, 1:2]
    gz = gt_ref[0, :, 2:3]
    gn = gx * gx + gy * gy + gz * gz
    px = pred_ref[0, 0:1, :]
    py = pred_ref[0, 1:2, :]
    pz = pred_ref[0, 2:3, :]
    pn = px * px + py * py + pz * pz
    ts = px.shape[1]
    d = (gn + pn) - 2.0 * (gx * px + gy * py + gz * pz)

    dcol = jnp.min(d, axis=1, keepdims=True)

    @pl.when(t == 0)
    def _():
        dmin_ref[...] = dcol

    @pl.when(t != 0)
    def _():
        dmin_ref[...] = jnp.minimum(dmin_ref[...], dcol)

    minv = jnp.min(d, axis=0, keepdims=True)
    iota0 = lax.broadcasted_iota(jnp.int32, d.shape, 0)
    idxm = jnp.min(jnp.where(d == minv, iota0, _NGT), axis=0, keepdims=True)
    idx_ref[0] = idxm

    lane = lax.broadcasted_iota(jnp.int32, (1, ts), 1) + t * ts
    s2 = jnp.sum(jnp.where(lane < NS, minv, 0.0))

    first = jnp.logical_and(b == 0, t == 0)

    @pl.when(first)
    def _():
        sd2_ref[0, 0] = s2

    @pl.when(jnp.logical_not(first))
    def _():
        sd2_ref[0, 0] = sd2_ref[0, 0] + s2

    @pl.when(jnp.logical_and(t == T - 1, b == 0))
    def _():
        sd1_ref[0, 0] = jnp.sum(dmin_ref[...])

    @pl.when(jnp.logical_and(t == T - 1, b != 0))
    def _():
        sd1_ref[0, 0] = sd1_ref[0, 0] + jnp.sum(dmin_ref[...])


def _chamfer(gt_points, pred_t, NS):
    ns_pad = pred_t.shape[-1]
    T = ns_pad // _TS
    body = functools.partial(_chamfer_body, T, NS)
    return pl.pallas_call(
        body,
        grid=(_B, T),
        in_specs=[
            pl.BlockSpec((1, _NGT, 3), lambda b, t: (b, 0, 0)),
            pl.BlockSpec((1, 3, _TS), lambda b, t: (b, 0, t)),
        ],
        out_specs=[
            pl.BlockSpec(memory_space=pltpu.SMEM),
            pl.BlockSpec(memory_space=pltpu.SMEM),
            pl.BlockSpec((1, 1, _TS), lambda b, t: (b, 0, t)),
        ],
        out_shape=[
            jax.ShapeDtypeStruct((1, 1), jnp.float32),
            jax.ShapeDtypeStruct((1, 1), jnp.float32),
            jax.ShapeDtypeStruct((_B, 1, ns_pad), jnp.int32),
        ],
        scratch_shapes=[pltpu.VMEM((_NGT, 1), jnp.float32)],
    )(gt_points, pred_t)


def _sc_body(p0, b0, p1, b1, p2, b2, nrm, i2_0, i2_1, i2_2,
             e00, e10, e01, e11, e02, e12,
             ln0, ln1, ln2, cn0, cn1, cn2,
             tri0, tri1, tri2, parts,
             px, py, pz, bx, by, bz, nx, ny, nz, i2v,
             e0v, e1v, lnv, cntv, av, cv, n2v, accv, sem):
    cid = lax.axis_index("c")
    sid = lax.axis_index("s")
    wid = sid * 2 + cid

    zero16 = jnp.zeros((16,), jnp.float32)
    for q in range(5):
        accv[pl.ds(q * 16, 16)] = zero16

    preds = (p0, p1, p2)
    befs = (b0, b1, b2)
    i2s = (i2_0, i2_1, i2_2)
    e0s = (e00, e01, e02)
    e1s = (e10, e11, e12)
    lns = (ln0, ln1, ln2)
    cns = (cn0, cn1, cn2)
    tris = (tri0, tri1, tri2)

    for b in range(_B):
        hn = [pltpu.async_copy(nrm.at[pl.ds((b * 3 + 0) * _NGT, _NGT)], nx, sem),
              pltpu.async_copy(nrm.at[pl.ds((b * 3 + 1) * _NGT, _NGT)], ny, sem),
              pltpu.async_copy(nrm.at[pl.ds((b * 3 + 2) * _NGT, _NGT)], nz, sem)]
        for h in hn:
            h.wait()
        for lev in range(3):
            nsp = _NSP[lev]
            ech = _ECH[lev]
            vch = _VCH[lev]
            ebase = wid * ech
            vbase = wid * vch
            pb = (b * 3) * nsp
            hs = [
                pltpu.async_copy(preds[lev].at[pl.ds(pb, nsp)],
                                 px.at[pl.ds(0, nsp)], sem),
                pltpu.async_copy(preds[lev].at[pl.ds(pb + nsp, nsp)],
                                 py.at[pl.ds(0, nsp)], sem),
                pltpu.async_copy(preds[lev].at[pl.ds(pb + 2 * nsp, nsp)],
                                 pz.at[pl.ds(0, nsp)], sem),
                pltpu.async_copy(befs[lev].at[pl.ds(pb, nsp)],
                                 bx.at[pl.ds(0, nsp)], sem),
                pltpu.async_copy(befs[lev].at[pl.ds(pb + nsp, nsp)],
                                 by.at[pl.ds(0, nsp)], sem),
                pltpu.async_copy(befs[lev].at[pl.ds(pb + 2 * nsp, nsp)],
                                 bz.at[pl.ds(0, nsp)], sem),
                pltpu.async_copy(i2s[lev].at[pl.ds(b * nsp, nsp)],
                                 i2v.at[pl.ds(0, nsp)], sem),
                pltpu.async_copy(e0s[lev].at[pl.ds(ebase, ech)],
                                 e0v.at[pl.ds(0, ech)], sem),
                pltpu.async_copy(e1s[lev].at[pl.ds(ebase, ech)],
                                 e1v.at[pl.ds(0, ech)], sem),
                pltpu.async_copy(cns[lev].at[pl.ds(vbase, vch)],
                                 cntv.at[pl.ds(0, vch)], sem),
            ]
            for j in range(8):
                hs.append(pltpu.async_copy(
                    lns[lev].at[pl.ds(j * nsp + vbase, vch)],
                    lnv.at[j, pl.ds(0, vch)], sem))
            for h in hs:
                h.wait()

            def edge_iter(i, carry):
                off = i * 16
                e0 = e0v[pl.ds(off, 16)]
                e1 = e1v[pl.ds(off, 16)]
                dx = plsc.load_gather(px, [e0]) - plsc.load_gather(px, [e1])
                dy = plsc.load_gather(py, [e0]) - plsc.load_gather(py, [e1])
                dz = plsc.load_gather(pz, [e0]) - plsc.load_gather(pz, [e1])
                ni = plsc.load_gather(i2v, [e0])
                gx = plsc.load_gather(nx, [ni])
                gy = plsc.load_gather(ny, [ni])
                gz = plsc.load_gather(nz, [ni])
                av[pl.ds(off, 16)] = dx * dx + dy * dy + dz * dz
                cv[pl.ds(off, 16)] = dx * gx + dy * gy + dz * gz
                n2v[pl.ds(off, 16)] = gx * gx + gy * gy + gz * gz
                return carry

            lax.fori_loop(0, ech // 16, edge_iter, 0)
            nep = _NEP[lev]
            tb = (b * 3) * nep + ebase
            ho = [
                pltpu.async_copy(av.at[pl.ds(0, ech)],
                                 tris[lev].at[pl.ds(tb, ech)], sem),
                pltpu.async_copy(cv.at[pl.ds(0, ech)],
                                 tris[lev].at[pl.ds(tb + nep, ech)], sem),
                pltpu.async_copy(n2v.at[pl.ds(0, ech)],
                                 tris[lev].at[pl.ds(tb + 2 * nep, ech)], sem),
            ]

            _LAPQ = lev
            _MOVQ = 2 + lev if lev > 0 else -1

            def lap_iter(i, carry):
                off = i * 16
                g = vbase + off
                dxv = bx[pl.ds(g, 16)] - px[pl.ds(g, 16)]
                dyv = by[pl.ds(g, 16)] - py[pl.ds(g, 16)]
                dzv = bz[pl.ds(g, 16)] - pz[pl.ds(g, 16)]
                sx = jnp.zeros((16,), jnp.float32)
                sy = jnp.zeros((16,), jnp.float32)
                sz = jnp.zeros((16,), jnp.float32)
                for j in range(8):
                    nb = lnv[j, pl.ds(off, 16)]
                    vf = jnp.where(nb >= 0, 1.0, 0.0).astype(jnp.float32)
                    nbs = jnp.maximum(nb, 0)
                    sx = sx + (plsc.load_gather(bx, [nbs])
                               - plsc.load_gather(px, [nbs])) * vf
                    sy = sy + (plsc.load_gather(by, [nbs])
                               - plsc.load_gather(py, [nbs])) * vf
                    sz = sz + (plsc.load_gather(bz, [nbs])
                               - plsc.load_gather(pz, [nbs])) * vf
                cc = cntv[pl.ds(off, 16)]
                lx = dxv - sx / cc
                ly = dyv - sy / cc
                lz = dzv - sz / cc
                lo = _LAPQ * 16
                accv[pl.ds(lo, 16)] = (accv[pl.ds(lo, 16)]
                                       + lx * lx + ly * ly + lz * lz)
                if _MOVQ >= 0:
                    mo = _MOVQ * 16
                    accv[pl.ds(mo, 16)] = (accv[pl.ds(mo, 16)]
                                           + dxv * dxv + dyv * dyv + dzv * dzv)
                return carry

            lax.fori_loop(0, vch // 16, lap_iter, 0)
            for h in ho:
                h.wait()

    pltpu.sync_copy(accv, parts.at[pl.ds(wid * 80, 80)])


def _sc_gather(pred_ts, bef_ts, nrm_t, idx2s, e0s, e1s, lapns, cnts):
    mesh = plsc.VectorSubcoreMesh(core_axis_name="c", subcore_axis_name="s",
                                  num_cores=2, num_subcores=16)
    out_type = [
        jax.ShapeDtypeStruct((_B * 3 * _NEP[0],), jnp.float32),
        jax.ShapeDtypeStruct((_B * 3 * _NEP[1],), jnp.float32),
        jax.ShapeDtypeStruct((_B * 3 * _NEP[2],), jnp.float32),
        jax.ShapeDtypeStruct((_NW * 80,), jnp.float32),
    ]
    scratch = [
        pltpu.VMEM((2560,), jnp.float32),
        pltpu.VMEM((2560,), jnp.float32),
        pltpu.VMEM((2560,), jnp.float32),
        pltpu.VMEM((2560,), jnp.float32),
        pltpu.VMEM((2560,), jnp.float32),
        pltpu.VMEM((2560,), jnp.float32),
        pltpu.VMEM((2048,), jnp.float32),
        pltpu.VMEM((2048,), jnp.float32),
        pltpu.VMEM((2048,), jnp.float32),
        pltpu.VMEM((2560,), jnp.int32),
        pltpu.VMEM((256,), jnp.int32),
        pltpu.VMEM((256,), jnp.int32),
        pltpu.VMEM((8, 128), jnp.int32),
        pltpu.VMEM((128,), jnp.float32),
        pltpu.VMEM((256,), jnp.float32),
        pltpu.VMEM((256,), jnp.float32),
        pltpu.VMEM((256,), jnp.float32),
        pltpu.VMEM((80,), jnp.float32),
        pltpu.SemaphoreType.DMA,
    ]
    f = pl.kernel(_sc_body, out_type=out_type, mesh=mesh,
                  scratch_types=scratch,
                  compiler_params=pltpu.CompilerParams(
                      use_tc_tiling_on_sc=False,
                      needs_layout_passes=False))
    return f(pred_ts[0], bef_ts[0], pred_ts[1], bef_ts[1],
             pred_ts[2], bef_ts[2], nrm_t,
             idx2s[0], idx2s[1], idx2s[2],
             e0s[0], e1s[0], e0s[1], e1s[1], e0s[2], e1s[2],
             lapns[0], lapns[1], lapns[2], cnts[0], cnts[1], cnts[2])


def _bce_body(gt_ref, p_ref, out_ref):
    p = jnp.clip(p_ref[...], 1e-7, 1.0 - 1e-7)
    gt = gt_ref[...]
    out_ref[0, 0] = jnp.sum(gt * jnp.log(p) + (1.0 - gt) * jnp.log(1.0 - p))


def _bce(gt_img, pred_img):
    return pl.pallas_call(
        _bce_body,
        out_specs=pl.BlockSpec(memory_space=pltpu.SMEM),
        out_shape=jax.ShapeDtypeStruct((1, 1), jnp.float32),
    )(gt_img, pred_img)


def _combine_body(sd1_0, sd2_0, sd1_1, sd2_1, sd1_2, sd2_2,
                  tri0, tri1, tri2, parts, bs, *outs):
    sd1 = (sd1_0[0, 0], sd1_1[0, 0], sd1_2[0, 0])
    sd2 = (sd2_0[0, 0], sd2_1[0, 0], sd2_2[0, 0])
    tris = (tri0, tri1, tri2)
    chamfer = jnp.float32(0.0)
    edge = jnp.float32(0.0)
    normal = jnp.float32(0.0)
    lap = jnp.float32(0.0)
    move = jnp.float32(0.0)
    for i in range(3):
        ns = _NSL[i]
        ne = _NEL[i]
        a = tris[i][:, 0, :]
        c = tris[i][:, 1, :]
        n2 = tris[i][:, 2, :]
        cos = jnp.abs(c) / (jnp.maximum(jnp.sqrt(a), _EPS)
                            * jnp.maximum(jnp.sqrt(n2), _EPS))
        normal = normal + jnp.sum(cos) / (_B * ne)
        edge = edge + jnp.sum(a) / (_B * ne)
        chamfer = chamfer + _CHAMFER_W[i] * (
            sd1[i] / (_B * _NGT) + _CHAMFER_OPP_W * sd2[i] / (_B * ns))
        lap = lap + _LAP_CONST[i] * jnp.sum(parts[:, i, :]) / (_B * ns)
        if i > 0:
            move = move + _LAP_CONST[i] * jnp.sum(parts[:, 2 + i, :]) / (_B * ns)
    image = -bs[0, 0] / (_B * 3 * 224 * 224)
    loss = (chamfer + image * _RECONST_W + _LAP_W * lap + _MOVE_W * move
            + _EDGE_W * edge + _NORMAL_W * normal)
    vals = (loss, image, chamfer, edge, lap, move, normal)
    for r, v in zip(outs, vals):
        r[0, 0] = v


def _combine(sds, tris, parts, bs):
    smem = pl.BlockSpec(memory_space=pltpu.SMEM)
    in_specs = [smem] * 6 + [pl.BlockSpec(t.shape, lambda: (0, 0, 0))
                             for t in tris]
    in_specs += [pl.BlockSpec(parts.shape, lambda: (0, 0, 0)), smem]
    return pl.pallas_call(
        _combine_body,
        in_specs=in_specs,
        out_specs=[smem] * 7,
        out_shape=[jax.ShapeDtypeStruct((1, 1), jnp.float32)] * 7,
    )(*sds, *tris, parts, bs)


def kernel(gt_points, gt_normals, gt_images, pred_reconst,
           pred_coord_0, pred_coord_1, pred_coord_2,
           pred_before_0, pred_before_1, pred_before_2,
           edges_0, edges_1, edges_2,
           lap_idx_0, lap_idx_1, lap_idx_2):
    gt_p = gt_points.astype(jnp.float32)
    nrm_t = jnp.transpose(gt_normals, (0, 2, 1)).astype(jnp.float32)
    preds = (pred_coord_0, pred_coord_1, pred_coord_2)
    befs = (pred_before_0, pred_before_1, pred_before_2)
    edges = (edges_0, edges_1, edges_2)
    laps = (lap_idx_0, lap_idx_1, lap_idx_2)

    pred_ts, bef_ts, e0s, e1s, lapns, cnts = [], [], [], [], [], []
    sds = []
    idx2s = []
    for i in range(3):
        ns = _NSL[i]
        nsp = _NSP[i]
        nep = _NEP[i]
        pred_t = _pad_last(jnp.transpose(preds[i], (0, 2, 1)), nsp, _BIG)
        bef_t = _pad_last(jnp.transpose(befs[i], (0, 2, 1)), nsp, _BIG)
        e = edges[i].astype(jnp.int32)
        e0 = _pad_last(e[:, 0], nep, 0)
        e1 = _pad_last(e[:, 1], nep, 0)
        li = laps[i].astype(jnp.int32)
        lapn = _pad_last(jnp.transpose(li[:, :8], (1, 0)), nsp, -1)
        cnt = _pad_last(li[:, 9].astype(jnp.float32), nsp, 1.0)
        sd1, sd2, idx2 = _chamfer(gt_p, pred_t, ns)
        sds += [sd1, sd2]
        idx2s.append(idx2)
        pred_ts.append(pred_t)
        bef_ts.append(bef_t)
        e0s.append(e0)
        e1s.append(e1)
        lapns.append(lapn)
        cnts.append(cnt)

    pred_fs = [jnp.reshape(p, (-1,)) for p in pred_ts]
    bef_fs = [jnp.reshape(p, (-1,)) for p in bef_ts]
    nrm_f = jnp.reshape(nrm_t, (-1,))
    idx2_fs = [jnp.reshape(ix, (-1,)) for ix in idx2s]
    lapn_fs = [jnp.reshape(ln, (-1,)) for ln in lapns]
    tri0, tri1, tri2, parts = _sc_gather(pred_fs, bef_fs, nrm_f, idx2_fs,
                                         e0s, e1s, lapn_fs, cnts)
    tri0 = jnp.reshape(tri0, (_B, 3, _NEP[0]))
    tri1 = jnp.reshape(tri1, (_B, 3, _NEP[1]))
    tri2 = jnp.reshape(tri2, (_B, 3, _NEP[2]))
    parts = jnp.reshape(parts, (_NW, 5, 16))

    gt_img = jnp.reshape(gt_images, (1176, 512))
    p_img = jnp.reshape(pred_reconst, (1176, 512))
    bs = _bce(gt_img, p_img)

    outs = _combine(sds, (tri0, tri1, tri2), parts, bs)
    return tuple(jnp.reshape(o, ()) for o in outs)
